# baseline (speedup 1.0000x reference)
import functools

import jax
import jax.numpy as jnp
from jax import lax
from jax.experimental import pallas as pl
from jax.experimental.pallas import tpu as pltpu
from jax.experimental.pallas import tpu_sc as plsc

LANES = 16
PSTRIDE = 32


def _quantize(xv, n_levels):
    r = xv * jnp.float32(n_levels - 1)
    rh = r + jnp.float32(0.5)
    n = rh.astype(jnp.int32)
    tie = n.astype(jnp.float32) == rh
    odd = lax.rem(n, 2) == 1
    n = jnp.where(tie & odd, n - 1, n)
    return jnp.clip(n, 0, n_levels - 1)


def _sc_histograms(x, size, n_levels, num_workers):
    B = x.shape[0]
    S = size * size
    H = 2 * n_levels * PSTRIDE
    G = H // 128
    assert S % LANES == 0 and H % 128 == 0 and B % num_workers == 0
    b_per_w = B // num_workers
    n_chunks = S // LANES
    row_halves = (size + LANES - 1) // LANES

    mesh = plsc.VectorSubcoreMesh(core_axis_name="c", subcore_axis_name="s")

    @functools.partial(
        pl.kernel,
        out_type=jax.ShapeDtypeStruct((B, 8, 128), jnp.float32),
        mesh=mesh,
        compiler_params=pltpu.CompilerParams(needs_layout_passes=False),
        scratch_types=[
            pltpu.VMEM((size, size), jnp.float32),
            pltpu.VMEM((S,), jnp.int32),
            pltpu.VMEM((H,), jnp.float32),
        ],
    )
    def sc_kernel(x_hbm, out_hbm, ximg_v, lev_v, hist_v):
        wid = lax.axis_index("s") * 2 + lax.axis_index("c")
        iota = lax.iota(jnp.int32, LANES)
        ones = jnp.ones((LANES,), jnp.float32)
        zeros = jnp.zeros((LANES,), jnp.float32)
        for bi in range(b_per_w):
            b = wid * b_per_w + bi
            pltpu.sync_copy(x_hbm.at[b], ximg_v)

            @plsc.parallel_loop(0, H // LANES)
            def _zero(c):
                hist_v[pl.ds(c * LANES, LANES)] = zeros

            @plsc.parallel_loop(0, n_chunks)
            def _pass1(k):
                s = k * LANES + iota
                row = s // size
                col = lax.rem(s, size)
                xv = plsc.load_gather(ximg_v, [row, col])
                lev = _quantize(xv, n_levels)
                lev_v[pl.ds(k * LANES, LANES)] = lev
                plsc.addupdate_scatter(hist_v, [lev * PSTRIDE + col], ones)

            @plsc.parallel_loop(0, size * row_halves)
            def _pass2(t):
                j = t // row_halves
                h = lax.rem(t, row_halves)
                i = h * LANES + iota
                m = i < size
                lev = plsc.load_gather(lev_v, [i * size + j], mask=m)
                plsc.addupdate_scatter(
                    hist_v,
                    [n_levels * PSTRIDE + lev * PSTRIDE + i], ones,
                    mask=m)

            for g in range(G):
                pltpu.sync_copy(hist_v.at[pl.ds(g * 128, 128)],
                                out_hbm.at[b, g])

    return sc_kernel(x)


def _tc_combine(C, pos_x_w, pos_y_w, value_w, classify_wT):
    B = C.shape[0]
    size, D = pos_x_w.shape
    L = value_w.shape[0]
    n_cls = classify_wT.shape[1]
    G = 2 * L * PSTRIDE // 128

    def body(c_ref, px_ref, py_ref, vw_ref, cwt_ref, out_ref):
        pad = jnp.zeros((PSTRIDE - size, D), jnp.float32)
        pxp = jnp.concatenate([px_ref[...], pad], axis=0)
        pyp = jnp.concatenate([py_ref[...], pad], axis=0)
        parts = [vw_ref[l:l + 1, :] * pxp for l in range(L)]
        parts += [vw_ref[l:l + 1, :] * pyp for l in range(L)]
        U = jnp.concatenate(parts, axis=0)
        multiset = jnp.zeros((B, D), jnp.float32)
        for g in range(G):
            multiset += lax.dot_general(
                c_ref[:, g, :], U[g * 128:(g + 1) * 128, :],
                (((1,), (0,)), ((), ())),
                preferred_element_type=jnp.float32)
        enc = jnp.where(multiset > 0, jnp.float32(1.0), jnp.float32(-1.0))
        out_ref[...] = lax.dot_general(
            enc, cwt_ref[...], (((1,), (0,)), ((), ())),
            precision=lax.Precision.HIGHEST,
            preferred_element_type=jnp.float32)

    return pl.pallas_call(
        body,
        out_shape=jax.ShapeDtypeStruct((B, n_cls), jnp.float32),
    )(C, pos_x_w, pos_y_w, value_w, classify_wT)


def kernel(x, pos_x_w, pos_y_w, value_w, classify_w):
    size = pos_x_w.shape[0]
    n_levels = value_w.shape[0]
    info = plsc.get_sparse_core_info()
    num_workers = info.num_cores * info.num_subcores
    C = _sc_histograms(x, size, n_levels, num_workers)
    return _tc_combine(C, pos_x_w, pos_y_w, value_w, classify_w.T)


# speedup vs baseline: 1.0216x; 1.0216x over previous
import functools

import jax
import jax.numpy as jnp
from jax import lax
from jax.experimental import pallas as pl
from jax.experimental.pallas import tpu as pltpu
from jax.experimental.pallas import tpu_sc as plsc

LANES = 16
PSTRIDE = 32


def _quantize(xv, n_levels):
    r = xv * jnp.float32(n_levels - 1)
    rh = r + jnp.float32(0.5)
    n = rh.astype(jnp.int32)
    tie = n.astype(jnp.float32) == rh
    odd = lax.rem(n, 2) == 1
    n = jnp.where(tie & odd, n - 1, n)
    return jnp.clip(n, 0, n_levels - 1)


def _sc_histograms(x, size, n_levels, num_workers):
    B = x.shape[0]
    S = size * size
    H = 2 * n_levels * PSTRIDE
    G = H // 128
    assert S % LANES == 0 and H % 128 == 0 and B % num_workers == 0
    b_per_w = B // num_workers
    n_chunks = S // LANES
    row_halves = (size + LANES - 1) // LANES

    mesh = plsc.VectorSubcoreMesh(core_axis_name="c", subcore_axis_name="s")

    halfw = size - LANES

    @functools.partial(
        pl.kernel,
        out_type=jax.ShapeDtypeStruct((B, 8, 128), jnp.float32),
        mesh=mesh,
        compiler_params=pltpu.CompilerParams(needs_layout_passes=False),
        scratch_types=[
            pltpu.VMEM((b_per_w * S,), jnp.float32),
            pltpu.VMEM((b_per_w * S,), jnp.int32),
            pltpu.VMEM((b_per_w * H,), jnp.float32),
            pltpu.SemaphoreType.DMA,
        ],
    )
    def sc_kernel(x_hbm, out_hbm, ximg_v, lev_v, hist_v, sem):
        wid = lax.axis_index("s") * 2 + lax.axis_index("c")
        iota = lax.iota(jnp.int32, LANES)
        ones = jnp.ones((LANES,), jnp.float32)
        zeros = jnp.zeros((LANES,), jnp.float32)
        b0 = wid * b_per_w
        pltpu.sync_copy(x_hbm.at[pl.ds(b0 * S, b_per_w * S)], ximg_v)

        @plsc.parallel_loop(0, b_per_w * H // LANES)
        def _zero(c):
            hist_v[pl.ds(c * LANES, LANES)] = zeros

        @plsc.parallel_loop(0, b_per_w * size * 2)
        def _pass1(t):
            bi = t // (size * 2)
            r = lax.rem(t, size * 2)
            i = r // 2
            h = lax.rem(r, 2)
            m = iota >= (LANES - halfw) * h
            col = h * halfw + iota
            xv = ximg_v[pl.ds(bi * S + i * size + h * halfw, LANES)]
            lev = _quantize(xv, n_levels)
            lev_v[pl.ds(bi * S + i * size + h * halfw, LANES)] = lev
            plsc.addupdate_scatter(
                hist_v, [bi * H + lev * PSTRIDE + col], ones, mask=m)

        @plsc.parallel_loop(0, b_per_w * size * row_halves)
        def _pass2(t):
            bi = t // (size * row_halves)
            r = lax.rem(t, size * row_halves)
            j = r // row_halves
            h = lax.rem(r, row_halves)
            i = h * LANES + iota
            m = i < size
            lev = plsc.load_gather(lev_v, [bi * S + i * size + j], mask=m)
            plsc.addupdate_scatter(
                hist_v,
                [bi * H + n_levels * PSTRIDE + lev * PSTRIDE + i], ones,
                mask=m)

        for bi in range(b_per_w):
            for g in range(G):
                pltpu.sync_copy(hist_v.at[pl.ds(bi * H + g * 128, 128)],
                                out_hbm.at[b0 + bi, g])

    return sc_kernel(x.reshape(-1))


def _tc_combine(C, pos_x_w, pos_y_w, value_w, classify_wT):
    B = C.shape[0]
    size, D = pos_x_w.shape
    L = value_w.shape[0]
    n_cls = classify_wT.shape[1]
    G = 2 * L * PSTRIDE // 128

    def body(c_ref, px_ref, py_ref, vw_ref, cwt_ref, out_ref):
        pad = jnp.zeros((PSTRIDE - size, D), jnp.float32)
        pxp = jnp.concatenate([px_ref[...], pad], axis=0)
        pyp = jnp.concatenate([py_ref[...], pad], axis=0)
        parts = [vw_ref[l:l + 1, :] * pxp for l in range(L)]
        parts += [vw_ref[l:l + 1, :] * pyp for l in range(L)]
        U = jnp.concatenate(parts, axis=0)
        multiset = jnp.zeros((B, D), jnp.float32)
        for g in range(G):
            multiset += lax.dot_general(
                c_ref[:, g, :], U[g * 128:(g + 1) * 128, :],
                (((1,), (0,)), ((), ())),
                preferred_element_type=jnp.float32)
        enc = jnp.where(multiset > 0, jnp.float32(1.0), jnp.float32(-1.0))
        out_ref[...] = lax.dot_general(
            enc, cwt_ref[...], (((1,), (0,)), ((), ())),
            precision=lax.Precision.HIGHEST,
            preferred_element_type=jnp.float32)

    return pl.pallas_call(
        body,
        out_shape=jax.ShapeDtypeStruct((B, n_cls), jnp.float32),
    )(C, pos_x_w, pos_y_w, value_w, classify_wT)


def kernel(x, pos_x_w, pos_y_w, value_w, classify_w):
    size = pos_x_w.shape[0]
    n_levels = value_w.shape[0]
    info = plsc.get_sparse_core_info()
    num_workers = info.num_cores * info.num_subcores
    C = _sc_histograms(x, size, n_levels, num_workers)
    return _tc_combine(C, pos_x_w, pos_y_w, value_w, classify_w.T)


# speedup vs baseline: 1.0818x; 1.0589x over previous
import functools

import jax
import jax.numpy as jnp
from jax import lax
from jax.experimental import pallas as pl
from jax.experimental.pallas import tpu as pltpu
from jax.experimental.pallas import tpu_sc as plsc

LANES = 16
PSTRIDE = 32


def _quantize(xv, n_levels):
    r = xv * jnp.float32(n_levels - 1)
    rh = r + jnp.float32(0.5)
    n = rh.astype(jnp.int32)
    tie = n.astype(jnp.float32) == rh
    odd = lax.rem(n, 2) == 1
    n = jnp.where(tie & odd, n - 1, n)
    return jnp.clip(n, 0, n_levels - 1)


def _sc_histograms(x, size, n_levels, num_workers):
    B = x.shape[0]
    S = size * size
    H = 2 * n_levels * PSTRIDE
    G = H // 128
    assert S % LANES == 0 and H % 128 == 0 and B % num_workers == 0
    b_per_w = B // num_workers
    n_chunks = S // LANES
    row_halves = (size + LANES - 1) // LANES

    mesh = plsc.VectorSubcoreMesh(core_axis_name="c", subcore_axis_name="s")

    halfw = size - LANES

    @functools.partial(
        pl.kernel,
        out_type=jax.ShapeDtypeStruct((B, 8, 128), jnp.float32),
        mesh=mesh,
        compiler_params=pltpu.CompilerParams(needs_layout_passes=False),
        scratch_types=[
            pltpu.VMEM((size, size), jnp.float32),
            pltpu.VMEM((size, size), jnp.float32),
            pltpu.VMEM((b_per_w * S,), jnp.int32),
            pltpu.VMEM((b_per_w * H,), jnp.float32),
        ],
    )
    def sc_kernel(x_hbm, out_hbm, ximg0_v, ximg1_v, lev_v, hist_v):
        wid = lax.axis_index("s") * 2 + lax.axis_index("c")
        iota = lax.iota(jnp.int32, LANES)
        ones = jnp.ones((LANES,), jnp.float32)
        zeros = jnp.zeros((LANES,), jnp.float32)
        b0 = wid * b_per_w
        pltpu.sync_copy(x_hbm.at[b0], ximg0_v)
        pltpu.sync_copy(x_hbm.at[b0 + 1], ximg1_v)

        @plsc.parallel_loop(0, b_per_w * H // LANES)
        def _zero(c):
            hist_v[pl.ds(c * LANES, LANES)] = zeros

        @plsc.parallel_loop(0, size * 2)
        def _pass1(t):
            i = t // 2
            h = lax.rem(t, 2)
            m = iota >= (LANES - halfw) * h
            col = h * halfw + iota
            for bi, img in ((0, ximg0_v), (1, ximg1_v)):
                xv = img[i, pl.ds(h * halfw, LANES)]
                lev = _quantize(xv, n_levels)
                lev_v[pl.ds(bi * S + i * size + h * halfw, LANES)] = lev
                plsc.addupdate_scatter(
                    hist_v, [bi * H + lev * PSTRIDE + col], ones, mask=m)

        @plsc.parallel_loop(0, size * row_halves)
        def _pass2(t):
            j = t // row_halves
            h = lax.rem(t, row_halves)
            i = h * LANES + iota
            m = i < size
            for bi in range(b_per_w):
                lev = plsc.load_gather(lev_v, [bi * S + i * size + j],
                                       mask=m)
                plsc.addupdate_scatter(
                    hist_v,
                    [bi * H + n_levels * PSTRIDE + lev * PSTRIDE + i],
                    ones, mask=m)

        for bi in range(b_per_w):
            for g in range(G):
                pltpu.sync_copy(hist_v.at[pl.ds(bi * H + g * 128, 128)],
                                out_hbm.at[b0 + bi, g])

    return sc_kernel(x)


def _tc_combine(C, pos_x_w, pos_y_w, value_w, classify_w):
    B = C.shape[0]
    size, D = pos_x_w.shape
    L = value_w.shape[0]
    n_cls = classify_w.shape[0]
    G = 2 * L * PSTRIDE // 128

    def body(c_ref, px_ref, py_ref, vw_ref, cw_ref, out_ref):
        pad = jnp.zeros((PSTRIDE - size, D), jnp.float32)
        pxp = jnp.concatenate([px_ref[...], pad], axis=0)
        pyp = jnp.concatenate([py_ref[...], pad], axis=0)
        parts = [vw_ref[l:l + 1, :] * pxp for l in range(L)]
        parts += [vw_ref[l:l + 1, :] * pyp for l in range(L)]
        U = jnp.concatenate(parts, axis=0)
        multiset = jnp.zeros((B, D), jnp.float32)
        for g in range(G):
            multiset += lax.dot_general(
                c_ref[:, g, :], U[g * 128:(g + 1) * 128, :],
                (((1,), (0,)), ((), ())),
                preferred_element_type=jnp.float32)
        enc = jnp.where(multiset > 0, jnp.float32(1.0), jnp.float32(-1.0))
        out_ref[...] = lax.dot_general(
            cw_ref[...], enc, (((1,), (1,)), ((), ())),
            precision=lax.Precision.HIGHEST,
            preferred_element_type=jnp.float32)

    return pl.pallas_call(
        body,
        out_shape=jax.ShapeDtypeStruct((n_cls, B), jnp.float32),
    )(C, pos_x_w, pos_y_w, value_w, classify_w)


def kernel(x, pos_x_w, pos_y_w, value_w, classify_w):
    size = pos_x_w.shape[0]
    n_levels = value_w.shape[0]
    info = plsc.get_sparse_core_info()
    num_workers = info.num_cores * info.num_subcores
    C = _sc_histograms(x, size, n_levels, num_workers)
    return _tc_combine(C, pos_x_w, pos_y_w, value_w, classify_w).T


# speedup vs baseline: 1.1031x; 1.0197x over previous
import functools

import jax
import jax.numpy as jnp
from jax import lax
from jax.experimental import pallas as pl
from jax.experimental.pallas import tpu as pltpu
from jax.experimental.pallas import tpu_sc as plsc

LANES = 16
PSTRIDE = 32


def _quantize(xv, n_levels):
    r = xv * jnp.float32(n_levels - 1)
    rh = r + jnp.float32(0.5)
    n = rh.astype(jnp.int32)
    tie = n.astype(jnp.float32) == rh
    odd = lax.rem(n, 2) == 1
    n = jnp.where(tie & odd, n - 1, n)
    return jnp.clip(n, 0, n_levels - 1)


def _sc_histograms(x, size, n_levels, num_workers):
    B = x.shape[0]
    S = size * size
    H = 2 * n_levels * PSTRIDE
    G = H // 128
    assert S % LANES == 0 and H % 128 == 0 and B % num_workers == 0
    b_per_w = B // num_workers
    n_chunks = S // LANES
    row_halves = (size + LANES - 1) // LANES

    mesh = plsc.VectorSubcoreMesh(core_axis_name="c", subcore_axis_name="s")

    halfw = size - LANES

    @functools.partial(
        pl.kernel,
        out_type=jax.ShapeDtypeStruct((B, 8, 128), jnp.float32),
        mesh=mesh,
        compiler_params=pltpu.CompilerParams(needs_layout_passes=False),
        scratch_types=[
            pltpu.VMEM((b_per_w * S,), jnp.float32),
            pltpu.VMEM((b_per_w * H,), jnp.float32),
        ],
    )
    def sc_kernel(x_hbm, out_hbm, ximg_v, hist_v):
        wid = lax.axis_index("s") * 2 + lax.axis_index("c")
        iota = lax.iota(jnp.int32, LANES)
        ones = jnp.ones((LANES,), jnp.float32)
        zeros = jnp.zeros((LANES,), jnp.float32)
        b0 = wid * b_per_w
        pltpu.sync_copy(x_hbm.at[pl.ds(b0 * S, b_per_w * S)], ximg_v)

        @plsc.parallel_loop(0, b_per_w * H // LANES)
        def _zero(c):
            hist_v[pl.ds(c * LANES, LANES)] = zeros

        @plsc.parallel_loop(0, b_per_w * size * 2)
        def _pass1(t):
            bi = t // (size * 2)
            r = lax.rem(t, size * 2)
            i = r // 2
            h = lax.rem(r, 2)
            m = iota >= (LANES - halfw) * h
            col = h * halfw + iota
            xv = ximg_v[pl.ds(bi * S + i * size + h * halfw, LANES)]
            lev = _quantize(xv, n_levels)
            plsc.addupdate_scatter(
                hist_v, [bi * H + lev * PSTRIDE + col], ones, mask=m)
            plsc.addupdate_scatter(
                hist_v,
                [bi * H + n_levels * PSTRIDE + lev * PSTRIDE + i], ones,
                mask=m)

        for bi in range(b_per_w):
            for g in range(G):
                pltpu.sync_copy(hist_v.at[pl.ds(bi * H + g * 128, 128)],
                                out_hbm.at[b0 + bi, g])

    return sc_kernel(x.reshape(-1))


def _tc_combine(C, pos_x_w, pos_y_w, value_w, classify_w):
    B = C.shape[0]
    size, D = pos_x_w.shape
    L = value_w.shape[0]
    n_cls = classify_w.shape[0]
    G = 2 * L * PSTRIDE // 128

    def body(c_ref, px_ref, py_ref, vw_ref, cw_ref, out_ref):
        pad = jnp.zeros((PSTRIDE - size, D), jnp.float32)
        pxp = jnp.concatenate([px_ref[...], pad], axis=0)
        pyp = jnp.concatenate([py_ref[...], pad], axis=0)
        parts = [vw_ref[l:l + 1, :] * pxp for l in range(L)]
        parts += [vw_ref[l:l + 1, :] * pyp for l in range(L)]
        U = jnp.concatenate(parts, axis=0)
        multiset = jnp.zeros((B, D), jnp.float32)
        for g in range(G):
            multiset += lax.dot_general(
                c_ref[:, g, :], U[g * 128:(g + 1) * 128, :],
                (((1,), (0,)), ((), ())),
                preferred_element_type=jnp.float32)
        enc = jnp.where(multiset > 0, jnp.float32(1.0), jnp.float32(-1.0))
        out_ref[...] = lax.dot_general(
            cw_ref[...], enc, (((1,), (1,)), ((), ())),
            precision=lax.Precision.HIGHEST,
            preferred_element_type=jnp.float32)

    return pl.pallas_call(
        body,
        out_shape=jax.ShapeDtypeStruct((n_cls, B), jnp.float32),
    )(C, pos_x_w, pos_y_w, value_w, classify_w)


def kernel(x, pos_x_w, pos_y_w, value_w, classify_w):
    size = pos_x_w.shape[0]
    n_levels = value_w.shape[0]
    info = plsc.get_sparse_core_info()
    num_workers = info.num_cores * info.num_subcores
    C = _sc_histograms(x, size, n_levels, num_workers)
    return _tc_combine(C, pos_x_w, pos_y_w, value_w, classify_w).T
